# Initial kernel scaffold; baseline (speedup 1.0000x reference)
import jax
import jax.numpy as jnp
from jax.experimental import pallas as pl


def kernel(input, target):
    raise NotImplementedError("write your pallas kernel here")


# speedup vs baseline: 63.7896x; 63.7896x over previous
import functools

import jax
import jax.numpy as jnp
from jax import lax
from jax.experimental import pallas as pl
from jax.experimental.pallas import tpu as pltpu
from jax.experimental.pallas import tpu_sc as plsc

NCLS = 21
B = 4
HW = 512 * 512
NW = 32
PIX_PER_W = HW // NW
CHUNK = 1024
NCHUNK = PIX_PER_W // CHUNK
LN2 = 0.6931471805599453
ACC_PAD = 16 * NCLS + 16


def _ln(p):
    i = plsc.bitcast(p, jnp.int32)
    e = (i >> 23) - 127
    m = plsc.bitcast((i & 0x007FFFFF) | 0x3F800000, jnp.float32)
    s = (m - 1.0) / (m + 1.0)
    u = s * s
    q = 1.0 / 3.0 + u * (1.0 / 5.0 + u * (1.0 / 7.0 + u * (1.0 / 9.0)))
    lnm = (s + s) * (1.0 + u * q)
    return e.astype(jnp.float32) * LN2 + lnm


def _focal_main_body(in_hbm, t_hbm, s_out, f_out, in_v, t_v, s_acc, f_acc,
                     svec, fvec):
    wid = lax.axis_index("s") * 2 + lax.axis_index("c")
    iota = lax.iota(jnp.int32, 16)
    zeros = jnp.zeros((16,), jnp.float32)
    ones = jnp.ones((16,), jnp.float32)
    lane_base = iota * NCLS

    for k in range(ACC_PAD // 16):
        s_acc[pl.ds(k * 16, 16)] = zeros
        f_acc[pl.ds(k * 16, 16)] = zeros
    for k in range(128 // 16):
        svec[pl.ds(k * 16, 16)] = zeros
        fvec[pl.ds(k * 16, 16)] = zeros

    def chunk_body(ci, carry):
        base = wid * PIX_PER_W + ci * CHUNK
        pltpu.sync_copy(t_hbm.at[:, pl.ds(base, CHUNK)], t_v)
        pltpu.sync_copy(in_hbm.at[:, :, pl.ds(base, CHUNK)], in_v)

        def px_body(j, c2):
            off = j * 16
            jvec = off + iota
            tsum = jnp.zeros((16,), jnp.float32)
            ts = []
            for b in range(B):
                tb = t_v[b, pl.ds(off, 16)]
                bvec = jnp.full((16,), b, jnp.int32)
                pb = plsc.load_gather(in_v, [bvec, tb, jvec])
                pb = jnp.minimum(jnp.maximum(pb, 1e-5), 1.0)
                omp = 1.0 - pb
                tsum = tsum + omp * omp * (0.0 - _ln(pb))
                ts.append(tb)
            for b in range(B):
                idx = lane_base + ts[b]
                plsc.addupdate_scatter(s_acc, [idx], tsum)
                plsc.addupdate_scatter(f_acc, [idx], ones)
            return c2

        return lax.fori_loop(0, CHUNK // 16, px_body, carry)

    lax.fori_loop(0, NCHUNK, chunk_body, 0)

    acc_s0 = zeros
    acc_s1 = zeros
    acc_f0 = zeros
    acc_f1 = zeros
    for l in range(16):
        acc_s0 = acc_s0 + s_acc[pl.ds(l * NCLS, 16)]
        acc_s1 = acc_s1 + s_acc[pl.ds(l * NCLS + 5, 16)]
        acc_f0 = acc_f0 + f_acc[pl.ds(l * NCLS, 16)]
        acc_f1 = acc_f1 + f_acc[pl.ds(l * NCLS + 5, 16)]
    svec[pl.ds(5, 16)] = acc_s1
    svec[pl.ds(0, 16)] = acc_s0
    fvec[pl.ds(5, 16)] = acc_f1
    fvec[pl.ds(0, 16)] = acc_f0

    pltpu.sync_copy(svec, s_out.at[wid])
    pltpu.sync_copy(fvec, f_out.at[wid])


_focal_main = functools.partial(
    pl.kernel,
    out_type=[
        jax.ShapeDtypeStruct((NW, 128), jnp.float32),
        jax.ShapeDtypeStruct((NW, 128), jnp.float32),
    ],
    mesh=plsc.VectorSubcoreMesh(core_axis_name="c", subcore_axis_name="s"),
    scratch_types=[
        pltpu.VMEM((B, NCLS, CHUNK), jnp.float32),
        pltpu.VMEM((B, CHUNK), jnp.int32),
        pltpu.VMEM((ACC_PAD,), jnp.float32),
        pltpu.VMEM((ACC_PAD,), jnp.float32),
        pltpu.VMEM((128,), jnp.float32),
        pltpu.VMEM((128,), jnp.float32),
    ],
    compiler_params=pltpu.CompilerParams(needs_layout_passes=False),
)(_focal_main_body)


def _combine_body(s_ref, f_ref, o_ref):
    s = jnp.sum(s_ref[...], axis=0)
    f = jnp.sum(f_ref[...], axis=0)
    cw = 1.0 / jnp.log(1.1 + f * (1.0 / float(B * HW)))
    o_ref[...] = jnp.sum(cw * s).reshape(1, 1)


def kernel(input, target):
    x = input.reshape(B, NCLS, HW)
    t = target.astype(jnp.int32).reshape(B, HW)
    s_tab, f_tab = _focal_main(x, t)
    out = pl.pallas_call(
        _combine_body,
        out_shape=jax.ShapeDtypeStruct((1, 1), jnp.float32),
    )(s_tab, f_tab)
    return out[0, 0]


# speedup vs baseline: 186.3638x; 2.9215x over previous
import functools

import jax
import jax.numpy as jnp
from jax import lax
from jax.experimental import pallas as pl
from jax.experimental.pallas import tpu as pltpu
from jax.experimental.pallas import tpu_sc as plsc

NCLS = 21
B = 4
H = 512
W = 512
HW = H * W
NW = 32
ROWS_PER_W = H // NW
LN2 = 0.6931471805599453
ACC_PAD = 16 * NCLS + 16


def _ln(p):
    i = plsc.bitcast(p, jnp.int32)
    e = (i >> 23) - 127
    m = plsc.bitcast((i & 0x007FFFFF) | 0x3F800000, jnp.float32)
    s = (m - 1.0) / (m + 1.0)
    u = s * s
    q = 1.0 / 3.0 + u * (1.0 / 5.0 + u * (1.0 / 7.0 + u * (1.0 / 9.0)))
    lnm = (s + s) * (1.0 + u * q)
    return e.astype(jnp.float32) * LN2 + lnm


def _focal_main_body(in_hbm, t_hbm, s_out, f_out, in_v, t_v, s_acc, f_acc,
                     svec, fvec, sem0, sem1):
    wid = lax.axis_index("s") * 2 + lax.axis_index("c")
    h0 = wid * ROWS_PER_W
    iota = lax.iota(jnp.int32, 16)
    zeros = jnp.zeros((16,), jnp.float32)
    zeros_i = jnp.zeros((16,), jnp.int32)
    ones = jnp.ones((16,), jnp.float32)
    lane_base = iota * NCLS
    sems = [sem0, sem1]

    for k in range(ACC_PAD // 16):
        s_acc[pl.ds(k * 16, 16)] = zeros
        f_acc[pl.ds(k * 16, 16)] = zeros
    for k in range(128 // 16):
        svec[pl.ds(k * 16, 16)] = zeros
        fvec[pl.ds(k * 16, 16)] = zeros

    def start_row(r, slot):
        h = h0 + r
        pltpu.async_copy(
            in_hbm.at[:, :, pl.ds(h, 1), :], in_v.at[slot], sems[slot]
        )
        pltpu.async_copy(
            t_hbm.at[:, :, pl.ds(h, 1), :], t_v.at[slot], sems[slot]
        )

    def wait_row(slot):
        pltpu.make_async_copy(
            in_hbm.at[:, :, pl.ds(0, 1), :], in_v.at[slot], sems[slot]
        ).wait()
        pltpu.make_async_copy(
            t_hbm.at[:, :, pl.ds(0, 1), :], t_v.at[slot], sems[slot]
        ).wait()

    def compute_row(slot):
        def px_body(jj, c2):
            off = jj * 16
            wvec = off + iota
            tsum = jnp.zeros((16,), jnp.float32)
            ts = []
            for b in range(B):
                tb = t_v[slot, b, 0, 0, pl.ds(off, 16)]
                bvec = jnp.full((16,), b, jnp.int32)
                pb = plsc.load_gather(
                    in_v.at[slot], [bvec, tb, zeros_i, wvec]
                )
                pb = jnp.minimum(jnp.maximum(pb, 1e-5), 1.0)
                omp = 1.0 - pb
                tsum = tsum + omp * omp * (0.0 - _ln(pb))
                ts.append(tb)
            for b in range(B):
                idx = lane_base + ts[b]
                plsc.addupdate_scatter(s_acc, [idx], tsum)
                plsc.addupdate_scatter(f_acc, [idx], ones)
            return c2

        lax.fori_loop(0, W // 16, px_body, 0)

    start_row(0, 0)
    for r in range(ROWS_PER_W):
        slot = r % 2
        if r + 1 < ROWS_PER_W:
            start_row(r + 1, 1 - slot)
        wait_row(slot)
        compute_row(slot)

    acc_s0 = zeros
    acc_s1 = zeros
    acc_f0 = zeros
    acc_f1 = zeros
    for l in range(16):
        acc_s0 = acc_s0 + s_acc[pl.ds(l * NCLS, 16)]
        acc_s1 = acc_s1 + s_acc[pl.ds(l * NCLS + 5, 16)]
        acc_f0 = acc_f0 + f_acc[pl.ds(l * NCLS, 16)]
        acc_f1 = acc_f1 + f_acc[pl.ds(l * NCLS + 5, 16)]
    svec[pl.ds(5, 16)] = acc_s1
    svec[pl.ds(0, 16)] = acc_s0
    fvec[pl.ds(5, 16)] = acc_f1
    fvec[pl.ds(0, 16)] = acc_f0

    pltpu.sync_copy(svec, s_out.at[wid])
    pltpu.sync_copy(fvec, f_out.at[wid])


_focal_main = functools.partial(
    pl.kernel,
    out_type=[
        jax.ShapeDtypeStruct((NW, 128), jnp.float32),
        jax.ShapeDtypeStruct((NW, 128), jnp.float32),
    ],
    mesh=plsc.VectorSubcoreMesh(core_axis_name="c", subcore_axis_name="s"),
    scratch_types=[
        pltpu.VMEM((2, B, NCLS, 1, W), jnp.float32),
        pltpu.VMEM((2, B, 1, 1, W), jnp.int32),
        pltpu.VMEM((ACC_PAD,), jnp.float32),
        pltpu.VMEM((ACC_PAD,), jnp.float32),
        pltpu.VMEM((128,), jnp.float32),
        pltpu.VMEM((128,), jnp.float32),
        pltpu.SemaphoreType.DMA,
        pltpu.SemaphoreType.DMA,
    ],
    compiler_params=pltpu.CompilerParams(needs_layout_passes=False),
)(_focal_main_body)


def _combine_body(s_ref, f_ref, o_ref):
    s = jnp.sum(s_ref[...], axis=0)
    f = jnp.sum(f_ref[...], axis=0)
    cw = 1.0 / jnp.log(1.1 + f * (1.0 / float(B * HW)))
    o_ref[...] = jnp.sum(cw * s).reshape(1, 1)


def kernel(input, target):
    t = target.astype(jnp.int32)
    s_tab, f_tab = _focal_main(input, t)
    out = pl.pallas_call(
        _combine_body,
        out_shape=jax.ShapeDtypeStruct((1, 1), jnp.float32),
    )(s_tab, f_tab)
    return out[0, 0]


# speedup vs baseline: 207.0977x; 1.1113x over previous
import functools

import jax
import jax.numpy as jnp
from jax import lax
from jax.experimental import pallas as pl
from jax.experimental.pallas import tpu as pltpu
from jax.experimental.pallas import tpu_sc as plsc

NCLS = 21
B = 4
H = 512
W = 512
HW = H * W
NW = 32
ROWS_PER_W = H // NW
ACC_PAD = 16 * NCLS + 16

NLN2 = -0.6931471805599453
NC = (
    7.98674648083697e-07,
    -1.0000083662233654,
    0.4998235247953396,
    -0.3325310803861114,
    0.2552293761746607,
    -0.2203877740075587,
    0.1376623938583184,
)


def _neg_ln(p):
    i = plsc.bitcast(p, jnp.int32)
    e = (i - 0x3F3504F3) >> 23
    m = plsc.bitcast(i - (e << 23), jnp.float32)
    r = m - 1.0
    poly = jnp.full((16,), NC[6], jnp.float32)
    for c in NC[5::-1]:
        poly = poly * r + c
    return e.astype(jnp.float32) * NLN2 + poly


def _focal_main_body(in_hbm, t_hbm, s_out, f_out, in_v, t_v, s_acc, f_acc,
                     svec, fvec, sem0, sem1):
    wid = lax.axis_index("s") * 2 + lax.axis_index("c")
    h0 = wid * ROWS_PER_W
    iota = lax.iota(jnp.int32, 16)
    zeros = jnp.zeros((16,), jnp.float32)
    zeros_i = jnp.zeros((16,), jnp.int32)
    ones = jnp.ones((16,), jnp.float32)
    lane_base = iota * NCLS
    sems = [sem0, sem1]

    for k in range(ACC_PAD // 16):
        s_acc[pl.ds(k * 16, 16)] = zeros
        f_acc[pl.ds(k * 16, 16)] = zeros
    for k in range(128 // 16):
        svec[pl.ds(k * 16, 16)] = zeros
        fvec[pl.ds(k * 16, 16)] = zeros

    def start_row(r, slot):
        h = h0 + r
        for b in range(B):
            pltpu.async_copy(
                in_hbm.at[b, :, pl.ds(h, 1), :],
                in_v.at[slot, pl.ds(b * NCLS, NCLS)],
                sems[slot],
            )
        pltpu.async_copy(
            t_hbm.at[:, :, pl.ds(h, 1), :], t_v.at[slot], sems[slot]
        )

    def wait_row(slot):
        for b in range(B):
            pltpu.make_async_copy(
                in_hbm.at[b, :, pl.ds(0, 1), :],
                in_v.at[slot, pl.ds(b * NCLS, NCLS)],
                sems[slot],
            ).wait()
        pltpu.make_async_copy(
            t_hbm.at[:, :, pl.ds(0, 1), :], t_v.at[slot], sems[slot]
        ).wait()

    def do_vreg(slot, off):
        wvec = off + iota
        tsum = jnp.zeros((16,), jnp.float32)
        ts = []
        for b in range(B):
            tb = t_v[slot, b, 0, 0, pl.ds(off, 16)]
            ct = tb + (b * NCLS)
            pb = plsc.load_gather(in_v.at[slot], [ct, zeros_i, wvec])
            pb = jnp.maximum(pb, 1e-5)
            omp = 1.0 - pb
            tsum = tsum + omp * omp * _neg_ln(pb)
            ts.append(tb)
        for b in range(B):
            idx = lane_base + ts[b]
            plsc.addupdate_scatter(s_acc, [idx], tsum)
            plsc.addupdate_scatter(f_acc, [idx], ones)

    def compute_row(slot):
        def px_body(jj, c2):
            off = jj * 32
            do_vreg(slot, off)
            do_vreg(slot, off + 16)
            return c2

        lax.fori_loop(0, W // 32, px_body, 0)

    start_row(0, 0)

    def pair_body(pr, carry):
        r0 = pr * 2
        start_row(r0 + 1, 1)
        wait_row(0)
        compute_row(0)

        @pl.when(r0 + 2 < ROWS_PER_W)
        def _():
            start_row(r0 + 2, 0)

        wait_row(1)
        compute_row(1)
        return carry

    lax.fori_loop(0, ROWS_PER_W // 2, pair_body, 0)

    acc_s0 = zeros
    acc_s1 = zeros
    acc_f0 = zeros
    acc_f1 = zeros
    for l in range(16):
        acc_s0 = acc_s0 + s_acc[pl.ds(l * NCLS, 16)]
        acc_s1 = acc_s1 + s_acc[pl.ds(l * NCLS + 5, 16)]
        acc_f0 = acc_f0 + f_acc[pl.ds(l * NCLS, 16)]
        acc_f1 = acc_f1 + f_acc[pl.ds(l * NCLS + 5, 16)]
    svec[pl.ds(5, 16)] = acc_s1
    svec[pl.ds(0, 16)] = acc_s0
    fvec[pl.ds(5, 16)] = acc_f1
    fvec[pl.ds(0, 16)] = acc_f0

    pltpu.sync_copy(svec, s_out.at[wid])
    pltpu.sync_copy(fvec, f_out.at[wid])


_focal_main = functools.partial(
    pl.kernel,
    out_type=[
        jax.ShapeDtypeStruct((NW, 128), jnp.float32),
        jax.ShapeDtypeStruct((NW, 128), jnp.float32),
    ],
    mesh=plsc.VectorSubcoreMesh(core_axis_name="c", subcore_axis_name="s"),
    scratch_types=[
        pltpu.VMEM((2, B * NCLS, 1, W), jnp.float32),
        pltpu.VMEM((2, B, 1, 1, W), jnp.int32),
        pltpu.VMEM((ACC_PAD,), jnp.float32),
        pltpu.VMEM((ACC_PAD,), jnp.float32),
        pltpu.VMEM((128,), jnp.float32),
        pltpu.VMEM((128,), jnp.float32),
        pltpu.SemaphoreType.DMA,
        pltpu.SemaphoreType.DMA,
    ],
    compiler_params=pltpu.CompilerParams(needs_layout_passes=False),
)(_focal_main_body)


def _combine_body(s_ref, f_ref, o_ref):
    s = jnp.sum(s_ref[...], axis=0)
    f = jnp.sum(f_ref[...], axis=0)
    cw = 1.0 / jnp.log(1.1 + f * (1.0 / float(B * HW)))
    o_ref[...] = jnp.sum(cw * s).reshape(1, 1)


def kernel(input, target):
    t = target.astype(jnp.int32)
    s_tab, f_tab = _focal_main(input, t)
    out = pl.pallas_call(
        _combine_body,
        out_shape=jax.ShapeDtypeStruct((1, 1), jnp.float32),
    )(s_tab, f_tab)
    return out[0, 0]


# speedup vs baseline: 207.6044x; 1.0024x over previous
import functools

import jax
import jax.numpy as jnp
import numpy as np
from jax import lax
from jax.experimental import pallas as pl
from jax.experimental.pallas import tpu as pltpu
from jax.experimental.pallas import tpu_sc as plsc

NCLS = 21
B = 4
H = 512
W = 512
HW = H * W
NW = 32
ROWS_PER_W = H // NW
ACC_PAD = 16 * NCLS + 16

LN2 = 0.6931471805599453


def _ln_tables():
    k = np.arange(256)
    mk = 1.0 + k / 256.0
    mk1 = 1.0 + (k + 1) / 256.0
    bs = (np.log(mk1) - np.log(mk)) * 256.0
    as_ = np.log(mk) - bs * mk
    nb = (-bs).astype(np.float32)
    na2 = (-as_ + 127.0 * LN2).astype(np.float32)
    return na2, nb


_NA2_TAB, _NB_TAB = _ln_tables()


def _focal_main_body(in_hbm, t_hbm, na_hbm, nb_hbm, s_out, f_out, in_v, t_v,
                     na_v, nb_v, s_acc, f_acc, svec, fvec, sem0, sem1):
    wid = lax.axis_index("s") * 2 + lax.axis_index("c")
    h0 = wid * ROWS_PER_W
    iota = lax.iota(jnp.int32, 16)
    zeros = jnp.zeros((16,), jnp.float32)
    zeros_i = jnp.zeros((16,), jnp.int32)
    ones = jnp.ones((16,), jnp.float32)
    lane_base = iota * NCLS
    sems = [sem0, sem1]

    pltpu.sync_copy(na_hbm, na_v)
    pltpu.sync_copy(nb_hbm, nb_v)
    for k in range(ACC_PAD // 16):
        s_acc[pl.ds(k * 16, 16)] = zeros
        f_acc[pl.ds(k * 16, 16)] = zeros
    for k in range(128 // 16):
        svec[pl.ds(k * 16, 16)] = zeros
        fvec[pl.ds(k * 16, 16)] = zeros

    def start_row(r, slot):
        h = h0 + r
        for b in range(B):
            pltpu.async_copy(
                in_hbm.at[b, :, pl.ds(h, 1), :],
                in_v.at[slot, pl.ds(b * NCLS, NCLS)],
                sems[slot],
            )
        pltpu.async_copy(
            t_hbm.at[:, :, pl.ds(h, 1), :], t_v.at[slot], sems[slot]
        )

    def wait_row(slot):
        for b in range(B):
            pltpu.make_async_copy(
                in_hbm.at[b, :, pl.ds(0, 1), :],
                in_v.at[slot, pl.ds(b * NCLS, NCLS)],
                sems[slot],
            ).wait()
        pltpu.make_async_copy(
            t_hbm.at[:, :, pl.ds(0, 1), :], t_v.at[slot], sems[slot]
        ).wait()

    def do_vreg(slot, off):
        wvec = off + iota
        tsum = jnp.zeros((16,), jnp.float32)
        ts = []
        for b in range(B):
            tb = t_v[slot, b, 0, 0, pl.ds(off, 16)]
            ct = tb + (b * NCLS)
            pb = plsc.load_gather(in_v.at[slot], [ct, zeros_i, wvec])
            pb = jnp.maximum(pb, 1e-5)
            i = plsc.bitcast(pb, jnp.int32)
            kk = (i >> 15) & 0xFF
            m = plsc.bitcast((i & 0x007FFFFF) | 0x3F800000, jnp.float32)
            na = plsc.load_gather(na_v, [kk])
            nb = plsc.load_gather(nb_v, [kk])
            nlnp = na + nb * m - (i >> 23).astype(jnp.float32) * LN2
            omp = 1.0 - pb
            tsum = tsum + omp * omp * nlnp
            ts.append(tb)
        for b in range(B):
            idx = lane_base + ts[b]
            plsc.addupdate_scatter(s_acc, [idx], tsum)
            plsc.addupdate_scatter(f_acc, [idx], ones)

    def compute_row(slot):
        def px_body(jj, c2):
            off = jj * 32
            do_vreg(slot, off)
            do_vreg(slot, off + 16)
            return c2

        lax.fori_loop(0, W // 32, px_body, 0)

    start_row(0, 0)

    def pair_body(pr, carry):
        r0 = pr * 2
        start_row(r0 + 1, 1)
        wait_row(0)
        compute_row(0)

        @pl.when(r0 + 2 < ROWS_PER_W)
        def _():
            start_row(r0 + 2, 0)

        wait_row(1)
        compute_row(1)
        return carry

    lax.fori_loop(0, ROWS_PER_W // 2, pair_body, 0)

    acc_s0 = zeros
    acc_s1 = zeros
    acc_f0 = zeros
    acc_f1 = zeros
    for l in range(16):
        acc_s0 = acc_s0 + s_acc[pl.ds(l * NCLS, 16)]
        acc_s1 = acc_s1 + s_acc[pl.ds(l * NCLS + 5, 16)]
        acc_f0 = acc_f0 + f_acc[pl.ds(l * NCLS, 16)]
        acc_f1 = acc_f1 + f_acc[pl.ds(l * NCLS + 5, 16)]
    svec[pl.ds(5, 16)] = acc_s1
    svec[pl.ds(0, 16)] = acc_s0
    fvec[pl.ds(5, 16)] = acc_f1
    fvec[pl.ds(0, 16)] = acc_f0

    pltpu.sync_copy(svec, s_out.at[wid])
    pltpu.sync_copy(fvec, f_out.at[wid])


_focal_main = functools.partial(
    pl.kernel,
    out_type=[
        jax.ShapeDtypeStruct((NW, 128), jnp.float32),
        jax.ShapeDtypeStruct((NW, 128), jnp.float32),
    ],
    mesh=plsc.VectorSubcoreMesh(core_axis_name="c", subcore_axis_name="s"),
    scratch_types=[
        pltpu.VMEM((2, B * NCLS, 1, W), jnp.float32),
        pltpu.VMEM((2, B, 1, 1, W), jnp.int32),
        pltpu.VMEM((256,), jnp.float32),
        pltpu.VMEM((256,), jnp.float32),
        pltpu.VMEM((ACC_PAD,), jnp.float32),
        pltpu.VMEM((ACC_PAD,), jnp.float32),
        pltpu.VMEM((128,), jnp.float32),
        pltpu.VMEM((128,), jnp.float32),
        pltpu.SemaphoreType.DMA,
        pltpu.SemaphoreType.DMA,
    ],
    compiler_params=pltpu.CompilerParams(needs_layout_passes=False),
)(_focal_main_body)


def _combine_body(s_ref, f_ref, o_ref):
    s = jnp.sum(s_ref[...], axis=0)
    f = jnp.sum(f_ref[...], axis=0)
    cw = 1.0 / jnp.log(1.1 + f * (1.0 / float(B * HW)))
    o_ref[...] = jnp.sum(cw * s).reshape(1, 1)


def kernel(input, target):
    t = target.astype(jnp.int32)
    s_tab, f_tab = _focal_main(
        input, t, jnp.asarray(_NA2_TAB), jnp.asarray(_NB_TAB)
    )
    out = pl.pallas_call(
        _combine_body,
        out_shape=jax.ShapeDtypeStruct((1, 1), jnp.float32),
    )(s_tab, f_tab)
    return out[0, 0]
